# Initial kernel scaffold; baseline (speedup 1.0000x reference)
import jax
import jax.numpy as jnp
from jax.experimental import pallas as pl


def kernel(points, features, grouped_idx, Wpos, bpos, Wg, bg, Wo, bo, gamma, beta):
    raise NotImplementedError("write your pallas kernel here")


# speedup vs baseline: 12.1652x; 12.1652x over previous
import functools

import jax
import jax.numpy as jnp
from jax import lax
from jax.experimental import pallas as pl
from jax.experimental.pallas import tpu as pltpu
from jax.experimental.pallas import tpu_sc as plsc

N = 10000
K = 32
C = 128
NW = 32
CPC = 2
CPT = 314
NPAD = NW * CPT
NCHUNK = CPT // CPC
IDXC = CPC * K
FCH = C // 16
NPAD16 = NPAD + 16


def _prep_body(feats, ptsB, ptsA, wcomb, wg, bg, q_out, p_out, wcg_out):
    wcg = jnp.dot(wcomb[:], wg[:], preferred_element_type=jnp.float32)
    q_out[:] = (jnp.dot(feats[:], wg[:], preferred_element_type=jnp.float32)
                + jnp.dot(ptsB[:], wcg, preferred_element_type=jnp.float32))
    p_out[:] = jnp.dot(ptsA[:], wcg, preferred_element_type=jnp.float32) + bg[:]
    wcg_out[:] = wcg


_prep = pl.pallas_call(
    _prep_body,
    out_shape=(
        jax.ShapeDtypeStruct((NPAD, C), jnp.float32),
        jax.ShapeDtypeStruct((NPAD, C), jnp.float32),
        jax.ShapeDtypeStruct((8, C), jnp.float32),
    ),
)


def _rsqrt(d2):
    i = plsc.bitcast(d2, jnp.int32)
    i = jnp.int32(0x5F3759DF) - lax.shift_right_logical(i, 1)
    r = plsc.bitcast(i, jnp.float32)
    for _ in range(3):
        r = r * (1.5 - 0.5 * d2 * r * r)
    return r


def _sc_body(q_hbm, p_hbm, x_hbm, y_hbm, z_hbm, idx_hbm, w9_hbm, m_hbm,
             x_v, y_v, z_v, idx_v, qbuf, pbuf, w9v, obuf, sem):
    wid = lax.axis_index("s") * 2 + lax.axis_index("c")
    base_center = wid * CPT

    pltpu.sync_copy(x_hbm, x_v)
    pltpu.sync_copy(y_hbm, y_v)
    pltpu.sync_copy(z_hbm, z_v)
    pltpu.sync_copy(w9_hbm, w9v)

    def chunk_body(ci, carry):
        cbase = base_center + ci * CPC
        pltpu.sync_copy(idx_hbm.at[pl.ds(cbase * K, IDXC)], idx_v)
        pltpu.async_copy(q_hbm.at[idx_v], qbuf, sem).wait()
        pltpu.sync_copy(p_hbm.at[pl.ds(cbase, CPC)], pbuf)
        xc = x_v[pl.ds(cbase, 16)]
        yc = y_v[pl.ds(cbase, 16)]
        zc = z_v[pl.ds(cbase, 16)]
        for cc in range(CPC):
            xi, yi, zi = xc[cc], yc[cc], zc[cc]
            dist = []
            for g in range(2):
                jv = idx_v[pl.ds(cc * K + g * 16, 16)]
                dx = xi - plsc.load_gather(x_v, [jv])
                dy = yi - plsc.load_gather(y_v, [jv])
                dz = zi - plsc.load_gather(z_v, [jv])
                d2 = dx * dx + dy * dy + dz * dz
                dist.append(d2 * _rsqrt(jnp.maximum(d2, 1e-24)))
            pch = [pbuf[cc, pl.ds(f * 16, 16)] for f in range(FCH)]
            wch = [w9v[pl.ds(f * 16, 16)] for f in range(FCH)]
            acc = [jnp.zeros((16,), jnp.float32)] * FCH
            for k in range(K):
                ds_ = dist[k // 16][k % 16]
                row = cc * K + k
                for f in range(FCH):
                    v = qbuf[row, pl.ds(f * 16, 16)] + pch[f] + ds_ * wch[f]
                    acc[f] = jnp.maximum(acc[f], v)
            for f in range(FCH):
                obuf[cc, pl.ds(f * 16, 16)] = acc[f]
        pltpu.sync_copy(obuf, m_hbm.at[pl.ds(cbase, CPC)])
        return carry

    lax.fori_loop(0, NCHUNK, chunk_body, 0)


_sc_edge = functools.partial(
    pl.kernel,
    mesh=plsc.VectorSubcoreMesh(core_axis_name="c", subcore_axis_name="s"),
    out_type=jax.ShapeDtypeStruct((NPAD, C), jnp.float32),
    compiler_params=pltpu.CompilerParams(needs_layout_passes=False),
    scratch_types=[
        pltpu.VMEM((NPAD16,), jnp.float32),
        pltpu.VMEM((NPAD16,), jnp.float32),
        pltpu.VMEM((NPAD16,), jnp.float32),
        pltpu.VMEM((IDXC,), jnp.int32),
        pltpu.VMEM((IDXC, C), jnp.float32),
        pltpu.VMEM((CPC, C), jnp.float32),
        pltpu.VMEM((C,), jnp.float32),
        pltpu.VMEM((CPC, C), jnp.float32),
        pltpu.SemaphoreType.DMA,
    ],
)(_sc_body)


def _out_body(m, feats, wo, bo, gamma, beta, o_ref):
    z = (jnp.dot(m[:] + feats[:], wo[:], preferred_element_type=jnp.float32)
         + bo[:])
    mu = jnp.mean(z, axis=-1, keepdims=True)
    var = jnp.mean((z - mu) ** 2, axis=-1, keepdims=True)
    o_ref[:] = jnp.maximum(
        (z - mu) / jnp.sqrt(var + 1e-5) * gamma[:] + beta[:], 0.0)


_outk = pl.pallas_call(
    _out_body,
    out_shape=jax.ShapeDtypeStruct((N, C), jnp.float32),
)


def kernel(points, features, grouped_idx, Wpos, bpos, Wg, bg, Wo, bo, gamma, beta):
    pts = points[0]
    feats = features[0]
    idx = grouped_idx[0].astype(jnp.int32).reshape(-1)

    pad = NPAD - N
    feats_p = jnp.pad(feats, ((0, pad), (0, 0)))
    pts_p = jnp.pad(pts, ((0, pad), (0, 0)))
    idx_p = jnp.pad(idx, (0, pad * K))

    wcomb = jnp.concatenate([
        Wpos[0:3] + Wpos[6:9],
        Wpos[3:6] - Wpos[6:9],
        Wpos[9:10],
        bpos[None, :],
    ], axis=0)
    zeros = jnp.zeros((NPAD, 1), jnp.float32)
    ones = jnp.ones((NPAD, 1), jnp.float32)
    ptsA = jnp.concatenate([pts_p, zeros, zeros, zeros, zeros, ones], axis=1)
    ptsB = jnp.concatenate([zeros, zeros, zeros, pts_p, zeros, zeros], axis=1)

    q, p, wcg = _prep(feats_p, ptsB, ptsA, wcomb, Wg, bg[None, :])
    xyz_t = jnp.pad(pts_p.T, ((0, 0), (0, 16)))
    m = _sc_edge(q, p, xyz_t[0], xyz_t[1], xyz_t[2], idx_p, wcg[6])
    out = _outk(m[:N], feats, Wo, bo[None, :], gamma[None, :], beta[None, :])
    return out[None]
